# baseline (speedup 1.0000x reference)
import functools

import jax
import jax.numpy as jnp
from jax import lax
from jax.experimental import pallas as pl
from jax.experimental.pallas import tpu as pltpu
from jax.experimental.pallas import tpu_sc as plsc

NC = 2
NS = 16
NW = NC * NS
K = 128
L = 16


def _sc_mesh():
    return plsc.VectorSubcoreMesh(core_axis_name="c", subcore_axis_name="s")


@functools.lru_cache(maxsize=None)
def _make_deg_kernel(n_pad, n_chunks):
    rows_pt = n_pad // NS
    n_groups = n_chunks * K // L

    @functools.partial(
        pl.kernel,
        out_type=jax.ShapeDtypeStruct((NC, NS, rows_pt), jnp.float32),
        mesh=_sc_mesh(),
        compiler_params=pltpu.CompilerParams(
            use_tc_tiling_on_sc=False, needs_layout_passes=False),
        scratch_types=[
            pltpu.VMEM((n_chunks, K), jnp.int32),
            pltpu.VMEM((n_pad,), jnp.float32),
            pltpu.VMEM((rows_pt,), jnp.float32),
            pltpu.VMEM((rows_pt,), jnp.float32),
            pltpu.VMEM_SHARED((NS, n_pad), jnp.float32),
        ],
    )
    def deg_kernel(ei_hbm, out_hbm, dst_v, hist_v, sum_v, tmp_v, stage_sh):
        cid = lax.axis_index("c")
        sid = lax.axis_index("s")
        wid = cid * NS + sid
        pltpu.sync_copy(ei_hbm.at[1, wid], dst_v)

        zvec = jnp.zeros((L,), jnp.float32)

        def zfill(i, _):
            hist_v[pl.ds(i * L, L)] = zvec
            return 0

        lax.fori_loop(0, n_pad // L, zfill, 0)

        onevec = jnp.ones((L,), jnp.float32)
        gpc = K // L

        def hbody(g, _):
            j = g // gpc
            c = g - j * gpc
            idx = dst_v[j, pl.ds(c * L, L)]
            plsc.addupdate_scatter(hist_v, [idx], onevec)
            return 0

        lax.fori_loop(0, n_groups, hbody, 0)

        pltpu.sync_copy(hist_v, stage_sh.at[sid])
        plsc.subcore_barrier()

        base = sid * rows_pt
        pltpu.sync_copy(stage_sh.at[0, pl.ds(base, rows_pt)], sum_v)
        for t in range(1, NS):
            pltpu.sync_copy(stage_sh.at[t, pl.ds(base, rows_pt)], tmp_v)

            def abody(i, _):
                sl = pl.ds(i * L, L)
                sum_v[sl] = sum_v[sl] + tmp_v[sl]
                return 0

            lax.fori_loop(0, rows_pt // L, abody, 0)
        pltpu.sync_copy(sum_v, out_hbm.at[cid, sid])

    return deg_kernel


@functools.lru_cache(maxsize=None)
def _make_scatter_kernel(n_pad, n_chunks, h):
    rows_pt = n_pad // NS
    nz = h // L

    @functools.partial(
        pl.kernel,
        out_type=jax.ShapeDtypeStruct((NC, n_pad, h), jnp.float32),
        mesh=_sc_mesh(),
        compiler_params=pltpu.CompilerParams(use_tc_tiling_on_sc=False),
        scratch_types=[
            pltpu.VMEM((n_chunks, K), jnp.int32),
            pltpu.VMEM((n_chunks, K), jnp.int32),
            pltpu.VMEM((2, K, h), jnp.float32),
            pltpu.VMEM_SHARED((n_pad, h), jnp.float32),
            pltpu.VMEM_SHARED((n_pad, h), jnp.float32),
        ] + [pltpu.SemaphoreType.DMA] * 4,
    )
    def scatter_kernel(xp_hbm, ei_hbm, out_hbm,
                       src_v, dst_v, rows2_v, acc_sh, table_sh, *sems):
        cid = lax.axis_index("c")
        sid = lax.axis_index("s")
        wid = cid * NS + sid
        pltpu.sync_copy(ei_hbm.at[0, wid], src_v)
        pltpu.sync_copy(ei_hbm.at[1, wid], dst_v)

        pltpu.sync_copy(
            xp_hbm.at[pl.ds(sid * rows_pt, rows_pt)],
            table_sh.at[pl.ds(sid * rows_pt, rows_pt)],
        )

        zvec = jnp.zeros((L,), jnp.float32)
        zbuf = rows2_v.at[0]

        def zfill(i, _):
            r = i // nz
            c = i - r * nz
            zbuf[r, pl.ds(c * L, L)] = zvec
            return 0

        lax.fori_loop(0, K * nz, zfill, 0)

        def zacc(k_, _):
            pltpu.sync_copy(zbuf, acc_sh.at[pl.ds(sid * rows_pt + k_ * K, K)])
            return 0

        lax.fori_loop(0, rows_pt // K, zacc, 0)
        plsc.subcore_barrier()

        bufs = (rows2_v.at[0], rows2_v.at[1])
        gsems = sems[:2]
        pltpu.async_copy(table_sh.at[src_v.at[0]], bufs[0], gsems[0])

        def body(g, _):
            for a in range(2):
                j = g * 2 + a
                b = 1 - a
                jn = j + 1

                @pl.when(jn < n_chunks)
                def _():
                    pltpu.async_copy(
                        table_sh.at[src_v.at[jn]], bufs[b], gsems[b])

                pltpu.make_async_copy(
                    table_sh.at[src_v.at[j]], bufs[a], gsems[a]).wait()
                pltpu.sync_copy(bufs[a], acc_sh.at[dst_v.at[j]], add=True)
            return 0

        lax.fori_loop(0, n_chunks // 2, body, 0)
        plsc.subcore_barrier()
        pltpu.sync_copy(
            acc_sh.at[pl.ds(sid * rows_pt, rows_pt)],
            out_hbm.at[cid, pl.ds(sid * rows_pt, rows_pt)],
        )

    return scatter_kernel


def _tc_call(body, out_shapes):
    return pl.pallas_call(
        body,
        out_shape=out_shapes,
    )


def _t0_body(x_ref, w1_ref, xw_ref):
    xw_ref[...] = jnp.dot(
        x_ref[...], w1_ref[...], preferred_element_type=jnp.float32)


def _t1_body(n, degp_ref, xw_ref, dis_ref, xp1_ref):
    deg = degp_ref[0, :n, 0:1] + degp_ref[1, :n, 0:1] + 1.0
    dis = lax.rsqrt(deg)
    dis_ref[...] = dis
    n_pad, h = xp1_ref.shape
    xp1_ref[...] = jnp.concatenate(
        [xw_ref[...] * dis, jnp.zeros((n_pad - n, h), jnp.float32)], axis=0)


def _t2_body(n, p_ref, xp_ref, dis_ref, b_ref, w_ref, out_ref):
    h = b_ref.shape[0]
    s = p_ref[0, :n, :] + p_ref[1, :n, :] + xp_ref[0:n, :]
    dis = dis_ref[...]
    hid = jnp.maximum(dis * s + b_ref[...].reshape(1, h), 0.0)
    xw = jnp.dot(hid, w_ref[...], preferred_element_type=jnp.float32)
    n_pad, h2 = out_ref.shape
    out_ref[...] = jnp.concatenate(
        [xw * dis, jnp.zeros((n_pad - n, h2), jnp.float32)], axis=0)


def _t3_body(n, p_ref, xp_ref, dis_ref, b_ref, w_ref, blin_ref, out_ref):
    h = b_ref.shape[0]
    d_out = blin_ref.shape[0]
    s = p_ref[0, :n, :] + p_ref[1, :n, :] + xp_ref[0:n, :]
    hid = jnp.maximum(dis_ref[...] * s + b_ref[...].reshape(1, h), 0.0)
    out_ref[...] = (
        jnp.dot(hid, w_ref[...], preferred_element_type=jnp.float32)
        + blin_ref[...].reshape(1, d_out)
    )


def kernel(x, edge_index, W1, b1, W2, b2, Wlin, blin):
    n = x.shape[0]
    e = edge_index.shape[1]
    h1 = W1.shape[1]
    h2 = W2.shape[1]
    d_out = Wlin.shape[1]

    chunk_edges = NW * K * 4
    e_pad = ((e + chunk_edges - 1) // chunk_edges) * chunk_edges
    n_chunks = e_pad // (NW * K)
    n_pad = ((n + 1 + NS * K - 1) // (NS * K)) * (NS * K)

    ei = jnp.pad(edge_index, ((0, 0), (0, e_pad - e)), constant_values=n)
    ei_slab = ei.reshape(2, NW, n_chunks, K).astype(jnp.int32)

    xw1 = _tc_call(
        _t0_body, jax.ShapeDtypeStruct((n, h1), jnp.float32))(x, W1)

    degp = _make_deg_kernel(n_pad, n_chunks)(ei_slab)
    degp = degp.reshape(NC, n_pad, 1)

    dis, xp1 = _tc_call(
        functools.partial(_t1_body, n),
        (
            jax.ShapeDtypeStruct((n, 1), jnp.float32),
            jax.ShapeDtypeStruct((n_pad, h1), jnp.float32),
        ),
    )(degp, xw1)

    p1 = _make_scatter_kernel(n_pad, n_chunks, h1)(xp1, ei_slab)

    xp2 = _tc_call(
        functools.partial(_t2_body, n),
        jax.ShapeDtypeStruct((n_pad, h2), jnp.float32),
    )(p1, xp1, dis, b1, W2)

    p2 = _make_scatter_kernel(n_pad, n_chunks, h2)(xp2, ei_slab)

    out = _tc_call(
        functools.partial(_t3_body, n),
        jax.ShapeDtypeStruct((n, d_out), jnp.float32),
    )(p2, xp2, dis, b2, Wlin, blin)

    return out


# speedup vs baseline: 1.0017x; 1.0017x over previous
import functools

import jax
import jax.numpy as jnp
from jax import lax
from jax.experimental import pallas as pl
from jax.experimental.pallas import tpu as pltpu
from jax.experimental.pallas import tpu_sc as plsc

NC = 2
NS = 16
NW = NC * NS
K = 128
L = 16


def _sc_mesh():
    return plsc.VectorSubcoreMesh(core_axis_name="c", subcore_axis_name="s")


@functools.lru_cache(maxsize=None)
def _make_deg_kernel(n_pad, n_chunks):
    rows_pt = n_pad // NS
    n_groups = n_chunks * K // L

    @functools.partial(
        pl.kernel,
        out_type=jax.ShapeDtypeStruct((NC, NS, rows_pt), jnp.float32),
        mesh=_sc_mesh(),
        compiler_params=pltpu.CompilerParams(
            use_tc_tiling_on_sc=False, needs_layout_passes=False),
        scratch_types=[
            pltpu.VMEM((n_chunks, K), jnp.int32),
            pltpu.VMEM((n_pad,), jnp.float32),
            pltpu.VMEM((rows_pt,), jnp.float32),
            pltpu.VMEM((rows_pt,), jnp.float32),
            pltpu.VMEM_SHARED((NS, n_pad), jnp.float32),
        ],
    )
    def deg_kernel(ei_hbm, out_hbm, dst_v, hist_v, sum_v, tmp_v, stage_sh):
        cid = lax.axis_index("c")
        sid = lax.axis_index("s")
        wid = cid * NS + sid
        pltpu.sync_copy(ei_hbm.at[1, wid], dst_v)

        zvec = jnp.zeros((L,), jnp.float32)

        def zfill(i, _):
            hist_v[pl.ds(i * L, L)] = zvec
            return 0

        lax.fori_loop(0, n_pad // L, zfill, 0)

        onevec = jnp.ones((L,), jnp.float32)
        gpc = K // L

        def hbody(j, _):
            for c in range(gpc):
                idx = dst_v[j, pl.ds(c * L, L)]
                plsc.addupdate_scatter(hist_v, [idx], onevec)
            return 0

        lax.fori_loop(0, n_chunks, hbody, 0)

        pltpu.sync_copy(hist_v, stage_sh.at[sid])
        plsc.subcore_barrier()

        base = sid * rows_pt
        pltpu.sync_copy(stage_sh.at[0, pl.ds(base, rows_pt)], sum_v)
        for t in range(1, NS):
            pltpu.sync_copy(stage_sh.at[t, pl.ds(base, rows_pt)], tmp_v)

            def abody(i, _):
                sl = pl.ds(i * L, L)
                sum_v[sl] = sum_v[sl] + tmp_v[sl]
                return 0

            lax.fori_loop(0, rows_pt // L, abody, 0)
        pltpu.sync_copy(sum_v, out_hbm.at[cid, sid])

    return deg_kernel


@functools.lru_cache(maxsize=None)
def _make_scatter_kernel(n_pad, n_chunks, h):
    rows_pt = n_pad // NS
    nz = h // L

    @functools.partial(
        pl.kernel,
        out_type=jax.ShapeDtypeStruct((NC, n_pad, h), jnp.float32),
        mesh=_sc_mesh(),
        compiler_params=pltpu.CompilerParams(use_tc_tiling_on_sc=False),
        scratch_types=[
            pltpu.VMEM((n_chunks, K), jnp.int32),
            pltpu.VMEM((n_chunks, K), jnp.int32),
            pltpu.VMEM((2, K, h), jnp.float32),
            pltpu.VMEM_SHARED((n_pad, h), jnp.float32),
            pltpu.VMEM_SHARED((n_pad, h), jnp.float32),
        ] + [pltpu.SemaphoreType.DMA] * 4,
    )
    def scatter_kernel(xp_hbm, ei_hbm, out_hbm,
                       src_v, dst_v, rows2_v, acc_sh, table_sh, *sems):
        cid = lax.axis_index("c")
        sid = lax.axis_index("s")
        wid = cid * NS + sid
        pltpu.sync_copy(ei_hbm.at[0, wid], src_v)
        pltpu.sync_copy(ei_hbm.at[1, wid], dst_v)

        pltpu.sync_copy(
            xp_hbm.at[pl.ds(sid * rows_pt, rows_pt)],
            table_sh.at[pl.ds(sid * rows_pt, rows_pt)],
        )

        zvec = jnp.zeros((L,), jnp.float32)
        zbuf = rows2_v.at[0]

        def zfill(i, _):
            r = i // nz
            c = i - r * nz
            zbuf[r, pl.ds(c * L, L)] = zvec
            return 0

        lax.fori_loop(0, K * nz, zfill, 0)

        def zacc(k_, _):
            pltpu.sync_copy(zbuf, acc_sh.at[pl.ds(sid * rows_pt + k_ * K, K)])
            return 0

        lax.fori_loop(0, rows_pt // K, zacc, 0)
        plsc.subcore_barrier()

        bufs = (rows2_v.at[0], rows2_v.at[1])
        gsems = sems[:2]
        pltpu.async_copy(table_sh.at[src_v.at[0]], bufs[0], gsems[0])

        def body(g, _):
            for a in range(2):
                j = g * 2 + a
                b = 1 - a
                jn = j + 1

                @pl.when(jn < n_chunks)
                def _():
                    pltpu.async_copy(
                        table_sh.at[src_v.at[jn]], bufs[b], gsems[b])

                pltpu.make_async_copy(
                    table_sh.at[src_v.at[j]], bufs[a], gsems[a]).wait()
                pltpu.sync_copy(bufs[a], acc_sh.at[dst_v.at[j]], add=True)
            return 0

        lax.fori_loop(0, n_chunks // 2, body, 0)
        plsc.subcore_barrier()
        pltpu.sync_copy(
            acc_sh.at[pl.ds(sid * rows_pt, rows_pt)],
            out_hbm.at[cid, pl.ds(sid * rows_pt, rows_pt)],
        )

    return scatter_kernel


def _tc_call(body, out_shapes):
    return pl.pallas_call(
        body,
        out_shape=out_shapes,
    )


def _t0_body(x_ref, w1_ref, xw_ref):
    xw_ref[...] = jnp.dot(
        x_ref[...], w1_ref[...], preferred_element_type=jnp.float32)


def _t1_body(n, degp_ref, xw_ref, dis_ref, xp1_ref):
    deg = degp_ref[0, :n, 0:1] + degp_ref[1, :n, 0:1] + 1.0
    dis = lax.rsqrt(deg)
    dis_ref[...] = dis
    n_pad, h = xp1_ref.shape
    xp1_ref[...] = jnp.concatenate(
        [xw_ref[...] * dis, jnp.zeros((n_pad - n, h), jnp.float32)], axis=0)


def _t2_body(n, p_ref, xp_ref, dis_ref, b_ref, w_ref, out_ref):
    h = b_ref.shape[0]
    s = p_ref[0, :n, :] + p_ref[1, :n, :] + xp_ref[0:n, :]
    dis = dis_ref[...]
    hid = jnp.maximum(dis * s + b_ref[...].reshape(1, h), 0.0)
    xw = jnp.dot(hid, w_ref[...], preferred_element_type=jnp.float32)
    n_pad, h2 = out_ref.shape
    out_ref[...] = jnp.concatenate(
        [xw * dis, jnp.zeros((n_pad - n, h2), jnp.float32)], axis=0)


def _t3_body(n, p_ref, xp_ref, dis_ref, b_ref, w_ref, blin_ref, out_ref):
    h = b_ref.shape[0]
    d_out = blin_ref.shape[0]
    s = p_ref[0, :n, :] + p_ref[1, :n, :] + xp_ref[0:n, :]
    hid = jnp.maximum(dis_ref[...] * s + b_ref[...].reshape(1, h), 0.0)
    out_ref[...] = (
        jnp.dot(hid, w_ref[...], preferred_element_type=jnp.float32)
        + blin_ref[...].reshape(1, d_out)
    )


def kernel(x, edge_index, W1, b1, W2, b2, Wlin, blin):
    n = x.shape[0]
    e = edge_index.shape[1]
    h1 = W1.shape[1]
    h2 = W2.shape[1]
    d_out = Wlin.shape[1]

    chunk_edges = NW * K * 4
    e_pad = ((e + chunk_edges - 1) // chunk_edges) * chunk_edges
    n_chunks = e_pad // (NW * K)
    n_pad = ((n + 1 + NS * K - 1) // (NS * K)) * (NS * K)

    ei = jnp.pad(edge_index, ((0, 0), (0, e_pad - e)), constant_values=n)
    ei_slab = ei.reshape(2, NW, n_chunks, K).astype(jnp.int32)

    xw1 = _tc_call(
        _t0_body, jax.ShapeDtypeStruct((n, h1), jnp.float32))(x, W1)

    degp = _make_deg_kernel(n_pad, n_chunks)(ei_slab)
    degp = degp.reshape(NC, n_pad, 1)

    dis, xp1 = _tc_call(
        functools.partial(_t1_body, n),
        (
            jax.ShapeDtypeStruct((n, 1), jnp.float32),
            jax.ShapeDtypeStruct((n_pad, h1), jnp.float32),
        ),
    )(degp, xw1)

    p1 = _make_scatter_kernel(n_pad, n_chunks, h1)(xp1, ei_slab)

    xp2 = _tc_call(
        functools.partial(_t2_body, n),
        jax.ShapeDtypeStruct((n_pad, h2), jnp.float32),
    )(p1, xp1, dis, b1, W2)

    p2 = _make_scatter_kernel(n_pad, n_chunks, h2)(xp2, ei_slab)

    out = _tc_call(
        functools.partial(_t3_body, n),
        jax.ShapeDtypeStruct((n, d_out), jnp.float32),
    )(p2, xp2, dis, b2, Wlin, blin)

    return out


# speedup vs baseline: 1.0176x; 1.0159x over previous
import functools

import jax
import jax.numpy as jnp
from jax import lax
from jax.experimental import pallas as pl
from jax.experimental.pallas import tpu as pltpu
from jax.experimental.pallas import tpu_sc as plsc

NC = 2
NS = 16
NW = NC * NS
K = 128
L = 16


def _sc_mesh():
    return plsc.VectorSubcoreMesh(core_axis_name="c", subcore_axis_name="s")


@functools.lru_cache(maxsize=None)
def _make_deg_kernel(n_pad, n_chunks):
    rows_pt = n_pad // NS
    n_groups = n_chunks * K // L

    @functools.partial(
        pl.kernel,
        out_type=jax.ShapeDtypeStruct((NC, NS, rows_pt), jnp.float32),
        mesh=_sc_mesh(),
        compiler_params=pltpu.CompilerParams(
            use_tc_tiling_on_sc=False, needs_layout_passes=False),
        scratch_types=[
            pltpu.VMEM((n_chunks, K), jnp.int32),
            pltpu.VMEM((n_pad,), jnp.float32),
            pltpu.VMEM((rows_pt,), jnp.float32),
            pltpu.VMEM((NS, rows_pt), jnp.float32),
            pltpu.VMEM_SHARED((NS, n_pad), jnp.float32),
        ],
    )
    def deg_kernel(ei_hbm, out_hbm, dst_v, hist_v, sum_v, gath_v, stage_sh):
        cid = lax.axis_index("c")
        sid = lax.axis_index("s")
        wid = cid * NS + sid
        pltpu.sync_copy(ei_hbm.at[1, wid], dst_v)

        zvec = jnp.zeros((L,), jnp.float32)

        def zfill(i, _):
            hist_v[pl.ds(i * L, L)] = zvec
            return 0

        lax.fori_loop(0, n_pad // L, zfill, 0)

        onevec = jnp.ones((L,), jnp.float32)
        gpc = K // L

        def hbody(j, _):
            for c in range(gpc):
                idx = dst_v[j, pl.ds(c * L, L)]
                plsc.addupdate_scatter(hist_v, [idx], onevec)
            return 0

        lax.fori_loop(0, n_chunks, hbody, 0)

        pltpu.sync_copy(hist_v, stage_sh.at[sid])
        plsc.subcore_barrier()

        base = sid * rows_pt
        pltpu.sync_copy(stage_sh.at[:, pl.ds(base, rows_pt)], gath_v)

        def abody(i, _):
            sl = pl.ds(i * L, L)
            acc = gath_v[0, sl]
            for t in range(1, NS):
                acc = acc + gath_v[t, sl]
            sum_v[sl] = acc
            return 0

        lax.fori_loop(0, rows_pt // L, abody, 0)
        pltpu.sync_copy(sum_v, out_hbm.at[cid, sid])

    return deg_kernel


@functools.lru_cache(maxsize=None)
def _make_scatter_kernel(n_pad, n_chunks, h):
    rows_pt = n_pad // NS
    nz = h // L

    @functools.partial(
        pl.kernel,
        out_type=jax.ShapeDtypeStruct((NC, n_pad, h), jnp.float32),
        mesh=_sc_mesh(),
        compiler_params=pltpu.CompilerParams(use_tc_tiling_on_sc=False),
        scratch_types=[
            pltpu.VMEM((n_chunks, K), jnp.int32),
            pltpu.VMEM((n_chunks, K), jnp.int32),
            pltpu.VMEM((2, K, h), jnp.float32),
            pltpu.VMEM_SHARED((n_pad, h), jnp.float32),
            pltpu.VMEM_SHARED((n_pad, h), jnp.float32),
        ] + [pltpu.SemaphoreType.DMA] * 4,
    )
    def scatter_kernel(xp_hbm, ei_hbm, out_hbm,
                       src_v, dst_v, rows2_v, acc_sh, table_sh, *sems):
        cid = lax.axis_index("c")
        sid = lax.axis_index("s")
        wid = cid * NS + sid
        pltpu.sync_copy(ei_hbm.at[0, wid], src_v)
        pltpu.sync_copy(ei_hbm.at[1, wid], dst_v)

        pltpu.sync_copy(
            xp_hbm.at[pl.ds(sid * rows_pt, rows_pt)],
            table_sh.at[pl.ds(sid * rows_pt, rows_pt)],
        )

        zvec = jnp.zeros((L,), jnp.float32)
        zbuf = rows2_v.at[0]

        def zfill(i, _):
            r = i // nz
            c = i - r * nz
            zbuf[r, pl.ds(c * L, L)] = zvec
            return 0

        lax.fori_loop(0, K * nz, zfill, 0)

        def zacc(k_, _):
            pltpu.sync_copy(zbuf, acc_sh.at[pl.ds(sid * rows_pt + k_ * K, K)])
            return 0

        lax.fori_loop(0, rows_pt // K, zacc, 0)
        plsc.subcore_barrier()

        bufs = (rows2_v.at[0], rows2_v.at[1])
        gsems = sems[:2]
        pltpu.async_copy(table_sh.at[src_v.at[0]], bufs[0], gsems[0])

        def body(g, _):
            for a in range(2):
                j = g * 2 + a
                b = 1 - a
                jn = j + 1

                @pl.when(jn < n_chunks)
                def _():
                    pltpu.async_copy(
                        table_sh.at[src_v.at[jn]], bufs[b], gsems[b])

                pltpu.make_async_copy(
                    table_sh.at[src_v.at[j]], bufs[a], gsems[a]).wait()
                pltpu.sync_copy(bufs[a], acc_sh.at[dst_v.at[j]], add=True)
            return 0

        lax.fori_loop(0, n_chunks // 2, body, 0)
        plsc.subcore_barrier()
        pltpu.sync_copy(
            acc_sh.at[pl.ds(sid * rows_pt, rows_pt)],
            out_hbm.at[cid, pl.ds(sid * rows_pt, rows_pt)],
        )

    return scatter_kernel


def _tc_call(body, out_shapes):
    return pl.pallas_call(
        body,
        out_shape=out_shapes,
    )


def _t0_body(x_ref, w1_ref, xw_ref):
    xw_ref[...] = jnp.dot(
        x_ref[...], w1_ref[...], preferred_element_type=jnp.float32)


def _t1_body(n, degp_ref, xw_ref, dis_ref, xp1_ref):
    deg = degp_ref[0, :n, 0:1] + degp_ref[1, :n, 0:1] + 1.0
    dis = lax.rsqrt(deg)
    dis_ref[...] = dis
    n_pad, h = xp1_ref.shape
    xp1_ref[...] = jnp.concatenate(
        [xw_ref[...] * dis, jnp.zeros((n_pad - n, h), jnp.float32)], axis=0)


def _t2_body(n, p_ref, xp_ref, dis_ref, b_ref, w_ref, out_ref):
    h = b_ref.shape[0]
    s = p_ref[0, :n, :] + p_ref[1, :n, :] + xp_ref[0:n, :]
    dis = dis_ref[...]
    hid = jnp.maximum(dis * s + b_ref[...].reshape(1, h), 0.0)
    xw = jnp.dot(hid, w_ref[...], preferred_element_type=jnp.float32)
    n_pad, h2 = out_ref.shape
    out_ref[...] = jnp.concatenate(
        [xw * dis, jnp.zeros((n_pad - n, h2), jnp.float32)], axis=0)


def _t3_body(n, p_ref, xp_ref, dis_ref, b_ref, w_ref, blin_ref, out_ref):
    h = b_ref.shape[0]
    d_out = blin_ref.shape[0]
    s = p_ref[0, :n, :] + p_ref[1, :n, :] + xp_ref[0:n, :]
    hid = jnp.maximum(dis_ref[...] * s + b_ref[...].reshape(1, h), 0.0)
    out_ref[...] = (
        jnp.dot(hid, w_ref[...], preferred_element_type=jnp.float32)
        + blin_ref[...].reshape(1, d_out)
    )


def kernel(x, edge_index, W1, b1, W2, b2, Wlin, blin):
    n = x.shape[0]
    e = edge_index.shape[1]
    h1 = W1.shape[1]
    h2 = W2.shape[1]
    d_out = Wlin.shape[1]

    chunk_edges = NW * K * 4
    e_pad = ((e + chunk_edges - 1) // chunk_edges) * chunk_edges
    n_chunks = e_pad // (NW * K)
    n_pad = ((n + 1 + NS * K - 1) // (NS * K)) * (NS * K)

    ei = jnp.pad(edge_index, ((0, 0), (0, e_pad - e)), constant_values=n)
    ei_slab = ei.reshape(2, NW, n_chunks, K).astype(jnp.int32)

    xw1 = _tc_call(
        _t0_body, jax.ShapeDtypeStruct((n, h1), jnp.float32))(x, W1)

    degp = _make_deg_kernel(n_pad, n_chunks)(ei_slab)
    degp = degp.reshape(NC, n_pad, 1)

    dis, xp1 = _tc_call(
        functools.partial(_t1_body, n),
        (
            jax.ShapeDtypeStruct((n, 1), jnp.float32),
            jax.ShapeDtypeStruct((n_pad, h1), jnp.float32),
        ),
    )(degp, xw1)

    p1 = _make_scatter_kernel(n_pad, n_chunks, h1)(xp1, ei_slab)

    xp2 = _tc_call(
        functools.partial(_t2_body, n),
        jax.ShapeDtypeStruct((n_pad, h2), jnp.float32),
    )(p1, xp1, dis, b1, W2)

    p2 = _make_scatter_kernel(n_pad, n_chunks, h2)(xp2, ei_slab)

    out = _tc_call(
        functools.partial(_t3_body, n),
        jax.ShapeDtypeStruct((n, d_out), jnp.float32),
    )(p2, xp2, dis, b2, Wlin, blin)

    return out


# speedup vs baseline: 1.0283x; 1.0105x over previous
import functools

import jax
import jax.numpy as jnp
from jax import lax
from jax.experimental import pallas as pl
from jax.experimental.pallas import tpu as pltpu
from jax.experimental.pallas import tpu_sc as plsc

NC = 2
NS = 16
NW = NC * NS
K = 128
L = 16


def _sc_mesh():
    return plsc.VectorSubcoreMesh(core_axis_name="c", subcore_axis_name="s")


@functools.lru_cache(maxsize=None)
def _make_deg_kernel(n_pad, n_chunks):
    rows_pt = n_pad // NS

    @functools.partial(
        pl.kernel,
        out_type=jax.ShapeDtypeStruct((NC, n_pad, L), jnp.float32),
        mesh=_sc_mesh(),
        scratch_types=[
            pltpu.VMEM((n_chunks, K), jnp.int32),
            pltpu.VMEM((K, L), jnp.float32),
            pltpu.VMEM_SHARED((n_pad, L), jnp.float32),
        ],
    )
    def deg_kernel(ei_hbm, out_hbm, dst_v, ones_v, acc_sh):
        cid = lax.axis_index("c")
        sid = lax.axis_index("s")
        wid = cid * NS + sid
        pltpu.sync_copy(ei_hbm.at[1, wid], dst_v)

        zvec = jnp.zeros((L,), jnp.float32)

        def zfill(i, _):
            ones_v[i, :] = zvec
            return 0

        lax.fori_loop(0, K, zfill, 0)

        def zacc(k_, _):
            pltpu.sync_copy(ones_v, acc_sh.at[pl.ds(sid * rows_pt + k_ * K, K)])
            return 0

        lax.fori_loop(0, rows_pt // K, zacc, 0)

        onevec = jnp.ones((L,), jnp.float32)

        def ofill(i, _):
            ones_v[i, :] = onevec
            return 0

        lax.fori_loop(0, K, ofill, 0)
        plsc.subcore_barrier()

        def body(j, _):
            pltpu.sync_copy(ones_v, acc_sh.at[dst_v.at[j]], add=True)
            return 0

        lax.fori_loop(0, n_chunks, body, 0)
        plsc.subcore_barrier()
        pltpu.sync_copy(
            acc_sh.at[pl.ds(sid * rows_pt, rows_pt)],
            out_hbm.at[cid, pl.ds(sid * rows_pt, rows_pt)],
        )

    return deg_kernel


@functools.lru_cache(maxsize=None)
def _make_scatter_kernel(n_pad, n_chunks, h):
    rows_pt = n_pad // NS
    nz = h // L

    @functools.partial(
        pl.kernel,
        out_type=jax.ShapeDtypeStruct((NC, n_pad, h), jnp.float32),
        mesh=_sc_mesh(),
        compiler_params=pltpu.CompilerParams(use_tc_tiling_on_sc=False),
        scratch_types=[
            pltpu.VMEM((n_chunks, K), jnp.int32),
            pltpu.VMEM((n_chunks, K), jnp.int32),
            pltpu.VMEM((2, K, h), jnp.float32),
            pltpu.VMEM_SHARED((n_pad, h), jnp.float32),
            pltpu.VMEM_SHARED((n_pad, h), jnp.float32),
        ] + [pltpu.SemaphoreType.DMA] * 4,
    )
    def scatter_kernel(xp_hbm, ei_hbm, out_hbm,
                       src_v, dst_v, rows2_v, acc_sh, table_sh, *sems):
        cid = lax.axis_index("c")
        sid = lax.axis_index("s")
        wid = cid * NS + sid
        pltpu.sync_copy(ei_hbm.at[0, wid], src_v)
        pltpu.sync_copy(ei_hbm.at[1, wid], dst_v)

        pltpu.sync_copy(
            xp_hbm.at[pl.ds(sid * rows_pt, rows_pt)],
            table_sh.at[pl.ds(sid * rows_pt, rows_pt)],
        )

        zvec = jnp.zeros((L,), jnp.float32)
        zbuf = rows2_v.at[0]

        def zfill(i, _):
            r = i // nz
            c = i - r * nz
            zbuf[r, pl.ds(c * L, L)] = zvec
            return 0

        lax.fori_loop(0, K * nz, zfill, 0)

        def zacc(k_, _):
            pltpu.sync_copy(zbuf, acc_sh.at[pl.ds(sid * rows_pt + k_ * K, K)])
            return 0

        lax.fori_loop(0, rows_pt // K, zacc, 0)
        plsc.subcore_barrier()

        bufs = (rows2_v.at[0], rows2_v.at[1])
        gsems = sems[:2]
        pltpu.async_copy(table_sh.at[src_v.at[0]], bufs[0], gsems[0])

        def body(g, _):
            for a in range(2):
                j = g * 2 + a
                b = 1 - a
                jn = j + 1

                @pl.when(jn < n_chunks)
                def _():
                    pltpu.async_copy(
                        table_sh.at[src_v.at[jn]], bufs[b], gsems[b])

                pltpu.make_async_copy(
                    table_sh.at[src_v.at[j]], bufs[a], gsems[a]).wait()
                pltpu.sync_copy(bufs[a], acc_sh.at[dst_v.at[j]], add=True)
            return 0

        lax.fori_loop(0, n_chunks // 2, body, 0)
        plsc.subcore_barrier()
        pltpu.sync_copy(
            acc_sh.at[pl.ds(sid * rows_pt, rows_pt)],
            out_hbm.at[cid, pl.ds(sid * rows_pt, rows_pt)],
        )

    return scatter_kernel


def _tc_call(body, out_shapes):
    return pl.pallas_call(
        body,
        out_shape=out_shapes,
    )


def _t0_body(x_ref, w1_ref, xw_ref):
    xw_ref[...] = jnp.dot(
        x_ref[...], w1_ref[...], preferred_element_type=jnp.float32)


def _t1_body(n, degp_ref, xw_ref, dis_ref, xp1_ref):
    deg = degp_ref[0, :n, 0:1] + degp_ref[1, :n, 0:1] + 1.0
    dis = lax.rsqrt(deg)
    dis_ref[...] = dis
    n_pad, h = xp1_ref.shape
    xp1_ref[...] = jnp.concatenate(
        [xw_ref[...] * dis, jnp.zeros((n_pad - n, h), jnp.float32)], axis=0)


def _t2_body(n, p_ref, xp_ref, dis_ref, b_ref, w_ref, out_ref):
    h = b_ref.shape[0]
    s = p_ref[0, :n, :] + p_ref[1, :n, :] + xp_ref[0:n, :]
    dis = dis_ref[...]
    hid = jnp.maximum(dis * s + b_ref[...].reshape(1, h), 0.0)
    xw = jnp.dot(hid, w_ref[...], preferred_element_type=jnp.float32)
    n_pad, h2 = out_ref.shape
    out_ref[...] = jnp.concatenate(
        [xw * dis, jnp.zeros((n_pad - n, h2), jnp.float32)], axis=0)


def _t3_body(n, p_ref, xp_ref, dis_ref, b_ref, w_ref, blin_ref, out_ref):
    h = b_ref.shape[0]
    d_out = blin_ref.shape[0]
    s = p_ref[0, :n, :] + p_ref[1, :n, :] + xp_ref[0:n, :]
    hid = jnp.maximum(dis_ref[...] * s + b_ref[...].reshape(1, h), 0.0)
    out_ref[...] = (
        jnp.dot(hid, w_ref[...], preferred_element_type=jnp.float32)
        + blin_ref[...].reshape(1, d_out)
    )


def kernel(x, edge_index, W1, b1, W2, b2, Wlin, blin):
    n = x.shape[0]
    e = edge_index.shape[1]
    h1 = W1.shape[1]
    h2 = W2.shape[1]
    d_out = Wlin.shape[1]

    chunk_edges = NW * K * 4
    e_pad = ((e + chunk_edges - 1) // chunk_edges) * chunk_edges
    n_chunks = e_pad // (NW * K)
    n_pad = ((n + 1 + NS * K - 1) // (NS * K)) * (NS * K)

    ei = jnp.pad(edge_index, ((0, 0), (0, e_pad - e)), constant_values=n)
    ei_slab = ei.reshape(2, NW, n_chunks, K).astype(jnp.int32)

    xw1 = _tc_call(
        _t0_body, jax.ShapeDtypeStruct((n, h1), jnp.float32))(x, W1)

    degp = _make_deg_kernel(n_pad, n_chunks)(ei_slab)

    dis, xp1 = _tc_call(
        functools.partial(_t1_body, n),
        (
            jax.ShapeDtypeStruct((n, 1), jnp.float32),
            jax.ShapeDtypeStruct((n_pad, h1), jnp.float32),
        ),
    )(degp, xw1)

    p1 = _make_scatter_kernel(n_pad, n_chunks, h1)(xp1, ei_slab)

    xp2 = _tc_call(
        functools.partial(_t2_body, n),
        jax.ShapeDtypeStruct((n_pad, h2), jnp.float32),
    )(p1, xp1, dis, b1, W2)

    p2 = _make_scatter_kernel(n_pad, n_chunks, h2)(xp2, ei_slab)

    out = _tc_call(
        functools.partial(_t3_body, n),
        jax.ShapeDtypeStruct((n, d_out), jnp.float32),
    )(p2, xp2, dis, b2, Wlin, blin)

    return out
